# Initial kernel scaffold; baseline (speedup 1.0000x reference)
import jax
import jax.numpy as jnp
from jax.experimental import pallas as pl


def kernel(x1, edge_index1, edge_type1, x2, edge_index2, edge_type2, batch1, batch2, W_rel1, W_root1, b1, W_rel2, W_root2, b2, Wm1, bm1, Wm2, bm2, Wm3, bm3):
    raise NotImplementedError("write your pallas kernel here")


# speedup vs baseline: 10.2704x; 10.2704x over previous
import functools

import jax
import jax.numpy as jnp
from jax import lax
from jax.experimental import pallas as pl
from jax.experimental.pallas import tpu as pltpu
from jax.experimental.pallas import tpu_sc as plsc

NC = 2
NS = 16
LN = 16
CK = 128

_f32 = jnp.float32
_i32 = jnp.int32


def _mesh():
    return plsc.VectorSubcoreMesh(core_axis_name="c", subcore_axis_name="s")


@functools.lru_cache(maxsize=None)
def _edge_prep(n, r, e_real, e_pad):
    cnt_sz = r * n + 1
    cnt_pad = -((-cnt_sz) // 128) * 128
    zspan = cnt_pad // NS
    pte = e_pad // NS
    half = e_pad // 2
    pto = half // NS
    dump = r * n

    def body(src_hbm, dst_hbm, typ_hbm, gboth_hbm, w_hbm,
             dstA, typA, srcB, dstB, typB, key2d, onesb, zb, cbuf,
             wbuf, g0b, g1b, cnt_sh):
        cid = lax.axis_index("c")
        sid = lax.axis_index("s")

        def fill_z(i, c):
            zb[pl.ds(i * LN, LN)] = jnp.zeros((LN,), _f32)
            return c
        lax.fori_loop(0, zspan // LN, fill_z, 0)

        def fill_o(i, c):
            onesb[pl.ds(i * LN, LN)] = jnp.ones((LN,), _f32)
            return c
        lax.fori_loop(0, CK // LN, fill_o, 0)

        pltpu.sync_copy(zb, cnt_sh.at[pl.ds(sid * zspan, zspan)])
        plsc.subcore_barrier()

        base = sid * pte
        pltpu.sync_copy(dst_hbm.at[pl.ds(base, pte)], dstA)
        pltpu.sync_copy(typ_hbm.at[pl.ds(base, pte)], typA)

        def keys(t, c):
            row = t // (CK // LN)
            col = (t % (CK // LN)) * LN
            d = dstA[pl.ds(t * LN, LN)]
            ty = typA[pl.ds(t * LN, LN)]
            eid = base + t * LN + lax.iota(_i32, LN)
            k = jnp.where(eid < e_real, ty * n + d, dump)
            key2d[row, pl.ds(col, LN)] = k
            return c
        lax.fori_loop(0, pte // LN, keys, 0)

        def scat(j, c):
            pltpu.sync_copy(onesb, cnt_sh.at[key2d.at[j]], add=True)
            return c
        lax.fori_loop(0, pte // CK, scat, 0)
        plsc.subcore_barrier()

        obase = cid * half + sid * pto
        pltpu.sync_copy(src_hbm.at[pl.ds(obase, pto)], srcB)
        pltpu.sync_copy(dst_hbm.at[pl.ds(obase, pto)], dstB)
        pltpu.sync_copy(typ_hbm.at[pl.ds(obase, pto)], typB)

        def keys2(t, c):
            row = t // (CK // LN)
            col = (t % (CK // LN)) * LN
            d = dstB[pl.ds(t * LN, LN)]
            ty = typB[pl.ds(t * LN, LN)]
            eid = obase + t * LN + lax.iota(_i32, LN)
            k = jnp.where(eid < e_real, ty * n + d, dump)
            key2d[row, pl.ds(col, LN)] = k
            return c
        lax.fori_loop(0, pto // LN, keys2, 0)

        def gat(j, c):
            pltpu.sync_copy(cnt_sh.at[key2d.at[j]], cbuf.at[pl.ds(j * CK, CK)])
            return c
        lax.fori_loop(0, pto // CK, gat, 0)

        def wcomp(t, c):
            cn = cbuf[pl.ds(t * LN, LN)]
            s = srcB[pl.ds(t * LN, LN)]
            ty = typB[pl.ds(t * LN, LN)]
            eid = obase + t * LN + lax.iota(_i32, LN)
            w = jnp.where(eid < e_real, 1.0 / jnp.maximum(cn, 1.0), 0.0)
            g0 = (ty * n + s) * 2
            wbuf[pl.ds(t * LN, LN)] = w
            g0b[pl.ds(t * LN, LN)] = g0
            g1b[pl.ds(t * LN, LN)] = g0 + 1
            return c
        lax.fori_loop(0, pto // LN, wcomp, 0)

        pltpu.sync_copy(wbuf, w_hbm.at[pl.ds(obase, pto)])
        pltpu.sync_copy(g0b, gboth_hbm.at[0, pl.ds(obase, pto)])
        pltpu.sync_copy(g1b, gboth_hbm.at[1, pl.ds(obase, pto)])

    return pl.kernel(
        body,
        out_type=(jax.ShapeDtypeStruct((2, e_pad), _i32),
                  jax.ShapeDtypeStruct((e_pad,), _f32)),
        mesh=_mesh(),
        scratch_types=[
            pltpu.VMEM((pte,), _i32),
            pltpu.VMEM((pte,), _i32),
            pltpu.VMEM((pto,), _i32),
            pltpu.VMEM((pto,), _i32),
            pltpu.VMEM((pto,), _i32),
            pltpu.VMEM((pte // CK, CK), _i32),
            pltpu.VMEM((CK,), _f32),
            pltpu.VMEM((zspan,), _f32),
            pltpu.VMEM((pto,), _f32),
            pltpu.VMEM((pto,), _f32),
            pltpu.VMEM((pto,), _i32),
            pltpu.VMEM((pto,), _i32),
            pltpu.VMEM_SHARED((cnt_pad,), _f32),
        ],
        name="rgcn_edge_prep",
    )


@functools.lru_cache(maxsize=None)
def _aggregate(n, dh, e_pad):
    pte = e_pad // NS
    ch = 80
    nch = -((-n) // ch)
    nkt = -((-nch) // NS)

    def body(ytab, root, g2d, dst2d, w_hbm, hout,
             gsel, dstv, wc, rows, acc_sh):
        cid = lax.axis_index("c")
        sid = lax.axis_index("s")

        def fill_z(t, c):
            row = t // (dh // LN)
            col = (t % (dh // LN)) * LN
            rows[row, pl.ds(col, LN)] = jnp.zeros((LN,), _f32)
            return c
        lax.fori_loop(0, ch * dh // LN, fill_z, 0)

        def zacc(k, c):
            cidx = sid + k * NS

            @pl.when(cidx < nch)
            def _():
                pltpu.sync_copy(rows.at[pl.ds(0, ch)],
                                acc_sh.at[pl.ds(cidx * ch, ch)])
            return c
        lax.fori_loop(0, nkt, zacc, 0)

        base_r = sid * (pte // CK)
        pltpu.sync_copy(g2d.at[cid, pl.ds(base_r, pte // CK)], gsel)
        pltpu.sync_copy(dst2d.at[pl.ds(base_r, pte // CK)], dstv)
        plsc.subcore_barrier()

        def chunk(j, c):
            pltpu.sync_copy(w_hbm.at[pl.ds(sid * pte + j * CK, CK)], wc)
            pltpu.sync_copy(ytab.at[gsel.at[j]], rows.at[pl.ds(0, CK)])

            def scale(gi, c2):
                wv16 = wc[pl.ds(gi * LN, LN)]
                for e16 in range(LN):
                    e = gi * LN + e16
                    wsp = jnp.full((LN,), wv16[e16], _f32)
                    for k in range(dh // LN):
                        rows[e, pl.ds(k * LN, LN)] = (
                            rows[e, pl.ds(k * LN, LN)] * wsp)
                return c2
            lax.fori_loop(0, CK // LN, scale, 0)

            pltpu.sync_copy(rows.at[pl.ds(0, CK)],
                            acc_sh.at[dstv.at[j]], add=True)
            return c
        lax.fori_loop(0, pte // CK, chunk, 0)
        plsc.subcore_barrier()

        def drain(k, c):
            cidx = sid + k * NS

            @pl.when(cidx < nch)
            def _():
                r0 = cidx * ch
                pltpu.sync_copy(acc_sh.at[pl.ds(r0, ch)],
                                rows.at[pl.ds(0, ch)])
                pltpu.sync_copy(root.at[cid, pl.ds(r0, ch)],
                                rows.at[pl.ds(ch, ch)])

                def cmb(t, c2):
                    row = t // (dh // LN)
                    col = (t % (dh // LN)) * LN
                    v = (rows[row, pl.ds(col, LN)]
                         + rows[ch + row, pl.ds(col, LN)])
                    rows[row, pl.ds(col, LN)] = jnp.maximum(v, 0.0)
                    return c2
                lax.fori_loop(0, ch * dh // LN, cmb, 0)

                pltpu.sync_copy(rows.at[pl.ds(0, ch)],
                                hout.at[cid, pl.ds(r0, ch)])
            return c
        lax.fori_loop(0, nkt, drain, 0)

    return pl.kernel(
        body,
        out_type=jax.ShapeDtypeStruct((2, n, dh), _f32),
        mesh=_mesh(),
        scratch_types=[
            pltpu.VMEM((pte // CK, CK), _i32),
            pltpu.VMEM((pte // CK, CK), _i32),
            pltpu.VMEM((CK,), _f32),
            pltpu.VMEM((max(CK, 2 * ch), dh), _f32),
            pltpu.VMEM_SHARED((n, dh), _f32),
        ],
        name="rgcn_aggregate",
    )


@functools.lru_cache(maxsize=None)
def _transform(n, d, nw, bn):
    dh = d // 2
    nb = -((-n) // bn)
    grid = (nb, nw)

    def body(h_ref, w_ref, b_ref, y_ref, r_ref):
        j = pl.program_id(1)
        hb = jnp.concatenate([h_ref[0], h_ref[1]], axis=1)
        acc = jnp.dot(hb, w_ref[0], preferred_element_type=_f32)

        @pl.when(j < nw - 1)
        def _():
            y_ref[0] = acc

        @pl.when(j == nw - 1)
        def _():
            o = acc + b_ref[...]
            r_ref[0] = o[:, :dh]
            r_ref[1] = o[:, dh:]

    return pl.pallas_call(
        body,
        grid=grid,
        in_specs=[
            pl.BlockSpec((2, bn, dh), lambda i, j: (0, i, 0)),
            pl.BlockSpec((1, d, d), lambda i, j: (j, 0, 0)),
            pl.BlockSpec((1, d), lambda i, j: (0, 0)),
        ],
        out_specs=[
            pl.BlockSpec((1, bn, d), lambda i, j: (jnp.minimum(j, nw - 2), i, 0)),
            pl.BlockSpec((2, bn, dh), lambda i, j: (0, i, 0)),
        ],
        out_shape=(jax.ShapeDtypeStruct(((nw - 1), n, d), _f32),
                   jax.ShapeDtypeStruct((2, n, dh), _f32)),
    )


@functools.lru_cache(maxsize=None)
def _pool_mlp(n, d, g, h_dim, bn):
    dh = d // 2
    nb = -((-n) // bn)
    grid = (nb,)

    def body(h1_ref, h2_ref, b1_ref, b2_ref, wm1_ref, bm1_ref,
             wm2_ref, bm2_ref, wm3_ref, bm3_ref, out_ref,
             acc1, acc2, cnt1, cnt2):
        i = pl.program_id(0)

        @pl.when(i == 0)
        def _():
            acc1[...] = jnp.zeros_like(acc1)
            acc2[...] = jnp.zeros_like(acc2)
            cnt1[...] = jnp.zeros_like(cnt1)
            cnt2[...] = jnp.zeros_like(cnt2)

        ids = lax.broadcasted_iota(_i32, (g, bn), 0)
        for h_ref, b_ref, acc, cnt in ((h1_ref, b1_ref, acc1, cnt1),
                                       (h2_ref, b2_ref, acc2, cnt2)):
            b = b_ref[0, 0, :]
            oh = (b[None, :] == ids).astype(_f32)
            hb = jnp.concatenate([h_ref[0], h_ref[1]], axis=1)
            acc[...] += jnp.dot(oh, hb, preferred_element_type=_f32)
            cnt[...] += jnp.broadcast_to(
                jnp.sum(oh, axis=1, keepdims=True), cnt.shape)

        @pl.when(i == nb - 1)
        def _():
            m1 = acc1[...] / jnp.maximum(cnt1[...][:, :1], 1.0)
            m2 = acc2[...] / jnp.maximum(cnt2[...][:, :1], 1.0)
            hcat = jnp.concatenate([m1, m2], axis=1)
            z = jnp.dot(hcat, wm1_ref[...], preferred_element_type=_f32)
            z = jnp.maximum(z + bm1_ref[...], 0.0)
            z = jnp.dot(z, wm2_ref[...], preferred_element_type=_f32)
            z = jnp.maximum(z + bm2_ref[...], 0.0)
            out_ref[...] = (jnp.dot(z, wm3_ref[...],
                                    preferred_element_type=_f32)
                            + bm3_ref[...])

    return pl.pallas_call(
        body,
        grid=grid,
        in_specs=[
            pl.BlockSpec((2, bn, dh), lambda i: (0, i, 0)),
            pl.BlockSpec((2, bn, dh), lambda i: (0, i, 0)),
            pl.BlockSpec((1, 1, bn), lambda i: (i, 0, 0)),
            pl.BlockSpec((1, 1, bn), lambda i: (i, 0, 0)),
            pl.BlockSpec((2 * d, h_dim), lambda i: (0, 0)),
            pl.BlockSpec((1, h_dim), lambda i: (0, 0)),
            pl.BlockSpec((h_dim, h_dim), lambda i: (0, 0)),
            pl.BlockSpec((1, h_dim), lambda i: (0, 0)),
            pl.BlockSpec((h_dim, 128), lambda i: (0, 0)),
            pl.BlockSpec((1, 128), lambda i: (0, 0)),
        ],
        out_specs=pl.BlockSpec((g, 128), lambda i: (0, 0)),
        out_shape=jax.ShapeDtypeStruct((g, 128), _f32),
        scratch_shapes=[
            pltpu.VMEM((g, d), _f32),
            pltpu.VMEM((g, d), _f32),
            pltpu.VMEM((g, 128), _f32),
            pltpu.VMEM((g, 128), _f32),
        ],
    )


def kernel(x1, edge_index1, edge_type1, x2, edge_index2, edge_type2,
           batch1, batch2, W_rel1, W_root1, b1, W_rel2, W_root2, b2,
           Wm1, bm1, Wm2, bm2, Wm3, bm3):
    n, d = x1.shape
    e = edge_index1.shape[1]
    num_l, r = W_rel1.shape[0], W_rel1.shape[1]
    h_dim = Wm1.shape[1]
    out_dim = Wm3.shape[1]
    g = 16
    dh = d // 2
    bn = 1024
    e_pad = -((-e) // (2 * NS * CK)) * (2 * NS * CK)

    prep = _edge_prep(n, r, e, e_pad)
    agg = _aggregate(n, dh, e_pad)
    trans = _transform(n, d, r + 1, bn)

    def branch(x, ei, et, w_rel, w_root, bias):
        src = ei[0]
        dst = ei[1]
        pad = e_pad - e
        src_p = jnp.pad(src, (0, pad))
        dst_p = jnp.pad(dst, (0, pad))
        typ_p = jnp.pad(et, (0, pad))
        gboth, w = prep(src_p, dst_p, typ_p)
        g2d = gboth.reshape(2, e_pad // CK, CK)
        dst2d = dst_p.reshape(e_pad // CK, CK)
        h = jnp.transpose(x.reshape(n, 2, dh), (1, 0, 2))
        for l in range(num_l):
            wall = jnp.concatenate([w_rel[l], w_root[l][None]], axis=0)
            yrel, root = trans(h, wall, bias[l].reshape(1, d))
            ytab = yrel.reshape(2 * r * n, dh)
            h = agg(ytab, root, g2d, dst2d, w)
        return h

    h1 = branch(x1, edge_index1, edge_type1, W_rel1, W_root1, b1)
    h2 = branch(x2, edge_index2, edge_type2, W_rel2, W_root2, b2)

    nb = -((-n) // bn)
    bpad = nb * bn - n
    b1_3d = jnp.pad(batch1, (0, bpad), constant_values=g).reshape(nb, 1, bn)
    b2_3d = jnp.pad(batch2, (0, bpad), constant_values=g).reshape(nb, 1, bn)
    wm3p = jnp.pad(Wm3, ((0, 0), (0, 128 - out_dim)))
    bm3p = jnp.pad(bm3, (0, 128 - out_dim)).reshape(1, 128)

    pool = _pool_mlp(n, d, g, h_dim, bn)
    out = pool(h1, h2, b1_3d, b2_3d, Wm1, bm1.reshape(1, h_dim),
               Wm2, bm2.reshape(1, h_dim), wm3p, bm3p)
    return out[:, :out_dim]


# speedup vs baseline: 13.0171x; 1.2674x over previous
import functools

import jax
import jax.numpy as jnp
from jax import lax
from jax.experimental import pallas as pl
from jax.experimental.pallas import tpu as pltpu
from jax.experimental.pallas import tpu_sc as plsc

NC = 2
NS = 16
LN = 16
CK = 128

_f32 = jnp.float32
_i32 = jnp.int32


def _mesh():
    return plsc.VectorSubcoreMesh(core_axis_name="c", subcore_axis_name="s")


@functools.lru_cache(maxsize=None)
def _edge_prep(n, r, e_real, e_pad):
    cnt_sz = r * n + 1
    cnt_pad = -((-cnt_sz) // 128) * 128
    zspan = cnt_pad // NS
    pte = e_pad // NS
    half = e_pad // 2
    pto = half // NS
    dump = r * n

    def body(src_hbm, dst_hbm, typ_hbm, gboth_hbm, w_hbm,
             dstA, typA, srcB, dstB, typB, key2d, onesb, zb, cbuf,
             wbuf, g0b, g1b, cnt_sh):
        cid = lax.axis_index("c")
        sid = lax.axis_index("s")

        def fill_z(i, c):
            zb[pl.ds(i * LN, LN)] = jnp.zeros((LN,), _f32)
            return c
        lax.fori_loop(0, zspan // LN, fill_z, 0)

        def fill_o(i, c):
            onesb[pl.ds(i * LN, LN)] = jnp.ones((LN,), _f32)
            return c
        lax.fori_loop(0, CK // LN, fill_o, 0)

        pltpu.sync_copy(zb, cnt_sh.at[pl.ds(sid * zspan, zspan)])
        plsc.subcore_barrier()

        base = sid * pte
        pltpu.sync_copy(dst_hbm.at[pl.ds(base, pte)], dstA)
        pltpu.sync_copy(typ_hbm.at[pl.ds(base, pte)], typA)

        def keys(t, c):
            row = t // (CK // LN)
            col = (t % (CK // LN)) * LN
            d = dstA[pl.ds(t * LN, LN)]
            ty = typA[pl.ds(t * LN, LN)]
            eid = base + t * LN + lax.iota(_i32, LN)
            k = jnp.where(eid < e_real, ty * n + d, dump)
            key2d[row, pl.ds(col, LN)] = k
            return c
        lax.fori_loop(0, pte // LN, keys, 0)

        def scat(j, c):
            pltpu.sync_copy(onesb, cnt_sh.at[key2d.at[j]], add=True)
            return c
        lax.fori_loop(0, pte // CK, scat, 0)
        plsc.subcore_barrier()

        obase = cid * half + sid * pto
        pltpu.sync_copy(src_hbm.at[pl.ds(obase, pto)], srcB)
        pltpu.sync_copy(dst_hbm.at[pl.ds(obase, pto)], dstB)
        pltpu.sync_copy(typ_hbm.at[pl.ds(obase, pto)], typB)

        def keys2(t, c):
            row = t // (CK // LN)
            col = (t % (CK // LN)) * LN
            d = dstB[pl.ds(t * LN, LN)]
            ty = typB[pl.ds(t * LN, LN)]
            eid = obase + t * LN + lax.iota(_i32, LN)
            k = jnp.where(eid < e_real, ty * n + d, dump)
            key2d[row, pl.ds(col, LN)] = k
            return c
        lax.fori_loop(0, pto // LN, keys2, 0)

        def gat(j, c):
            pltpu.sync_copy(cnt_sh.at[key2d.at[j]], cbuf.at[pl.ds(j * CK, CK)])
            return c
        lax.fori_loop(0, pto // CK, gat, 0)

        def wcomp(t, c):
            cn = cbuf[pl.ds(t * LN, LN)]
            s = srcB[pl.ds(t * LN, LN)]
            ty = typB[pl.ds(t * LN, LN)]
            eid = obase + t * LN + lax.iota(_i32, LN)
            w = jnp.where(eid < e_real, 1.0 / jnp.maximum(cn, 1.0), 0.0)
            g0 = (ty * n + s) * 2
            wbuf[pl.ds(t * LN, LN)] = w
            g0b[pl.ds(t * LN, LN)] = g0
            g1b[pl.ds(t * LN, LN)] = g0 + 1
            return c
        lax.fori_loop(0, pto // LN, wcomp, 0)

        pltpu.sync_copy(wbuf, w_hbm.at[pl.ds(obase, pto)])
        pltpu.sync_copy(g0b, gboth_hbm.at[0, pl.ds(obase, pto)])
        pltpu.sync_copy(g1b, gboth_hbm.at[1, pl.ds(obase, pto)])

    return pl.kernel(
        body,
        out_type=(jax.ShapeDtypeStruct((2, e_pad), _i32),
                  jax.ShapeDtypeStruct((e_pad,), _f32)),
        mesh=_mesh(),
        scratch_types=[
            pltpu.VMEM((pte,), _i32),
            pltpu.VMEM((pte,), _i32),
            pltpu.VMEM((pto,), _i32),
            pltpu.VMEM((pto,), _i32),
            pltpu.VMEM((pto,), _i32),
            pltpu.VMEM((pte // CK, CK), _i32),
            pltpu.VMEM((CK,), _f32),
            pltpu.VMEM((zspan,), _f32),
            pltpu.VMEM((pto,), _f32),
            pltpu.VMEM((pto,), _f32),
            pltpu.VMEM((pto,), _i32),
            pltpu.VMEM((pto,), _i32),
            pltpu.VMEM_SHARED((cnt_pad,), _f32),
        ],
        name="rgcn_edge_prep",
    )


@functools.lru_cache(maxsize=None)
def _aggregate(n, dh, e_pad):
    pte = e_pad // NS
    nck = pte // CK
    npair = nck // 2
    ch = 80
    nch = -((-n) // ch)
    nkt = -((-nch) // NS)

    def body(ytab, root, gboth, dst_hbm, w_hbm, hout,
             gsel, dstb, wcb, rows, gsem0, gsem1, ssem0, ssem1, acc_sh):
        cid = lax.axis_index("c")
        sid = lax.axis_index("s")

        def fill_z(t, c):
            row = t // (dh // LN)
            col = (t % (dh // LN)) * LN
            rows[row, pl.ds(col, LN)] = jnp.zeros((LN,), _f32)
            return c
        lax.fori_loop(0, ch * dh // LN, fill_z, 0)

        def zacc(k, c):
            cidx = sid + k * NS

            @pl.when(cidx < nch)
            def _():
                pltpu.sync_copy(rows.at[pl.ds(0, ch)],
                                acc_sh.at[pl.ds(cidx * ch, ch)])
            return c
        lax.fori_loop(0, nkt, zacc, 0)

        ebase = sid * pte
        pltpu.sync_copy(gboth.at[cid, pl.ds(ebase, pte)], gsel)
        plsc.subcore_barrier()

        def g_issue(ck, buf, sem):
            pltpu.async_copy(w_hbm.at[pl.ds(ebase + ck * CK, CK)],
                             wcb.at[buf], sem)
            pltpu.async_copy(dst_hbm.at[pl.ds(ebase + ck * CK, CK)],
                             dstb.at[buf], sem)
            pltpu.async_copy(ytab.at[gsel.at[pl.ds(ck * CK, CK)]],
                             rows.at[pl.ds(buf * CK, CK)], sem)

        def g_wait(buf, sem):
            pltpu.make_async_copy(w_hbm.at[pl.ds(0, CK)],
                                  wcb.at[buf], sem).wait()
            pltpu.make_async_copy(dst_hbm.at[pl.ds(0, CK)],
                                  dstb.at[buf], sem).wait()
            pltpu.make_async_copy(ytab.at[pl.ds(0, CK)],
                                  rows.at[pl.ds(buf * CK, CK)], sem).wait()

        def s_issue(ck, buf, sem):
            pltpu.async_copy(rows.at[pl.ds(buf * CK, CK)],
                             acc_sh.at[dstb.at[buf]], sem, add=True)

        def s_wait(buf, sem):
            pltpu.make_async_copy(ytab.at[pl.ds(0, CK)],
                                  rows.at[pl.ds(buf * CK, CK)], sem).wait()

        def scale(buf):
            def sg(gi, c2):
                wv16 = wcb[buf, pl.ds(gi * LN, LN)]
                for e16 in range(LN):
                    e = buf * CK + gi * LN + e16
                    wsp = jnp.full((LN,), wv16[e16], _f32)
                    for k in range(dh // LN):
                        rows[e, pl.ds(k * LN, LN)] = (
                            rows[e, pl.ds(k * LN, LN)] * wsp)
                return c2
            lax.fori_loop(0, CK // LN, sg, 0)

        g_issue(0, 0, gsem0)

        def pair(j2, c):
            c0 = j2 * 2

            @pl.when(j2 > 0)
            def _():
                s_wait(1, ssem1)
            g_issue(c0 + 1, 1, gsem1)
            g_wait(0, gsem0)
            scale(0)
            s_issue(c0, 0, ssem0)

            @pl.when(j2 + 1 < npair)
            def _():
                s_wait(0, ssem0)
                g_issue(c0 + 2, 0, gsem0)
            g_wait(1, gsem1)
            scale(1)
            s_issue(c0 + 1, 1, ssem1)
            return c
        lax.fori_loop(0, npair, pair, 0)
        s_wait(0, ssem0)
        s_wait(1, ssem1)
        plsc.subcore_barrier()

        def drain(k, c):
            cidx = sid + k * NS

            @pl.when(cidx < nch)
            def _():
                r0 = cidx * ch
                pltpu.sync_copy(acc_sh.at[pl.ds(r0, ch)],
                                rows.at[pl.ds(0, ch)])
                pltpu.sync_copy(root.at[cid, pl.ds(r0, ch)],
                                rows.at[pl.ds(ch, ch)])

                def cmb(t, c2):
                    row = t // (dh // LN)
                    col = (t % (dh // LN)) * LN
                    v = (rows[row, pl.ds(col, LN)]
                         + rows[ch + row, pl.ds(col, LN)])
                    rows[row, pl.ds(col, LN)] = jnp.maximum(v, 0.0)
                    return c2
                lax.fori_loop(0, ch * dh // LN, cmb, 0)

                pltpu.sync_copy(rows.at[pl.ds(0, ch)],
                                hout.at[cid, pl.ds(r0, ch)])
            return c
        lax.fori_loop(0, nkt, drain, 0)

    return pl.kernel(
        body,
        out_type=jax.ShapeDtypeStruct((2, n, dh), _f32),
        mesh=_mesh(),
        scratch_types=[
            pltpu.VMEM((pte,), _i32),
            pltpu.VMEM((2, CK), _i32),
            pltpu.VMEM((2, CK), _f32),
            pltpu.VMEM((2 * CK, dh), _f32),
            pltpu.SemaphoreType.DMA,
            pltpu.SemaphoreType.DMA,
            pltpu.SemaphoreType.DMA,
            pltpu.SemaphoreType.DMA,
            pltpu.VMEM_SHARED((n, dh), _f32),
        ],
        name="rgcn_aggregate",
    )


@functools.lru_cache(maxsize=None)
def _transform(n, d, nw, bn):
    dh = d // 2
    nb = -((-n) // bn)
    grid = (nb, nw)

    def body(h_ref, w_ref, b_ref, y_ref, r_ref):
        j = pl.program_id(1)
        hb = jnp.concatenate([h_ref[0], h_ref[1]], axis=1)
        acc = jnp.dot(hb, w_ref[0], preferred_element_type=_f32)

        @pl.when(j < nw - 1)
        def _():
            y_ref[0] = acc

        @pl.when(j == nw - 1)
        def _():
            o = acc + b_ref[...]
            r_ref[0] = o[:, :dh]
            r_ref[1] = o[:, dh:]

    return pl.pallas_call(
        body,
        grid=grid,
        in_specs=[
            pl.BlockSpec((2, bn, dh), lambda i, j: (0, i, 0)),
            pl.BlockSpec((1, d, d), lambda i, j: (j, 0, 0)),
            pl.BlockSpec((1, d), lambda i, j: (0, 0)),
        ],
        out_specs=[
            pl.BlockSpec((1, bn, d), lambda i, j: (jnp.minimum(j, nw - 2), i, 0)),
            pl.BlockSpec((2, bn, dh), lambda i, j: (0, i, 0)),
        ],
        out_shape=(jax.ShapeDtypeStruct(((nw - 1), n, d), _f32),
                   jax.ShapeDtypeStruct((2, n, dh), _f32)),
    )


@functools.lru_cache(maxsize=None)
def _pool_mlp(n, d, g, h_dim, bn):
    dh = d // 2
    nb = -((-n) // bn)
    grid = (nb,)

    def body(h1_ref, h2_ref, b1_ref, b2_ref, wm1_ref, bm1_ref,
             wm2_ref, bm2_ref, wm3_ref, bm3_ref, out_ref,
             acc1, acc2, cnt1, cnt2):
        i = pl.program_id(0)

        @pl.when(i == 0)
        def _():
            acc1[...] = jnp.zeros_like(acc1)
            acc2[...] = jnp.zeros_like(acc2)
            cnt1[...] = jnp.zeros_like(cnt1)
            cnt2[...] = jnp.zeros_like(cnt2)

        ids = lax.broadcasted_iota(_i32, (g, bn), 0)
        for h_ref, b_ref, acc, cnt in ((h1_ref, b1_ref, acc1, cnt1),
                                       (h2_ref, b2_ref, acc2, cnt2)):
            b = b_ref[0, 0, :]
            oh = (b[None, :] == ids).astype(_f32)
            hb = jnp.concatenate([h_ref[0], h_ref[1]], axis=1)
            acc[...] += jnp.dot(oh, hb, preferred_element_type=_f32)
            cnt[...] += jnp.broadcast_to(
                jnp.sum(oh, axis=1, keepdims=True), cnt.shape)

        @pl.when(i == nb - 1)
        def _():
            m1 = acc1[...] / jnp.maximum(cnt1[...][:, :1], 1.0)
            m2 = acc2[...] / jnp.maximum(cnt2[...][:, :1], 1.0)
            hcat = jnp.concatenate([m1, m2], axis=1)
            z = jnp.dot(hcat, wm1_ref[...], preferred_element_type=_f32)
            z = jnp.maximum(z + bm1_ref[...], 0.0)
            z = jnp.dot(z, wm2_ref[...], preferred_element_type=_f32)
            z = jnp.maximum(z + bm2_ref[...], 0.0)
            out_ref[...] = (jnp.dot(z, wm3_ref[...],
                                    preferred_element_type=_f32)
                            + bm3_ref[...])

    return pl.pallas_call(
        body,
        grid=grid,
        in_specs=[
            pl.BlockSpec((2, bn, dh), lambda i: (0, i, 0)),
            pl.BlockSpec((2, bn, dh), lambda i: (0, i, 0)),
            pl.BlockSpec((1, 1, bn), lambda i: (i, 0, 0)),
            pl.BlockSpec((1, 1, bn), lambda i: (i, 0, 0)),
            pl.BlockSpec((2 * d, h_dim), lambda i: (0, 0)),
            pl.BlockSpec((1, h_dim), lambda i: (0, 0)),
            pl.BlockSpec((h_dim, h_dim), lambda i: (0, 0)),
            pl.BlockSpec((1, h_dim), lambda i: (0, 0)),
            pl.BlockSpec((h_dim, 128), lambda i: (0, 0)),
            pl.BlockSpec((1, 128), lambda i: (0, 0)),
        ],
        out_specs=pl.BlockSpec((g, 128), lambda i: (0, 0)),
        out_shape=jax.ShapeDtypeStruct((g, 128), _f32),
        scratch_shapes=[
            pltpu.VMEM((g, d), _f32),
            pltpu.VMEM((g, d), _f32),
            pltpu.VMEM((g, 128), _f32),
            pltpu.VMEM((g, 128), _f32),
        ],
    )


def kernel(x1, edge_index1, edge_type1, x2, edge_index2, edge_type2,
           batch1, batch2, W_rel1, W_root1, b1, W_rel2, W_root2, b2,
           Wm1, bm1, Wm2, bm2, Wm3, bm3):
    n, d = x1.shape
    e = edge_index1.shape[1]
    num_l, r = W_rel1.shape[0], W_rel1.shape[1]
    h_dim = Wm1.shape[1]
    out_dim = Wm3.shape[1]
    g = 16
    dh = d // 2
    bn = 1024
    e_pad = -((-e) // (2 * NS * CK)) * (2 * NS * CK)

    prep = _edge_prep(n, r, e, e_pad)
    agg = _aggregate(n, dh, e_pad)
    trans = _transform(n, d, r + 1, bn)

    def branch(x, ei, et, w_rel, w_root, bias):
        src = ei[0]
        dst = ei[1]
        pad = e_pad - e
        src_p = jnp.pad(src, (0, pad))
        dst_p = jnp.pad(dst, (0, pad))
        typ_p = jnp.pad(et, (0, pad))
        gboth, w = prep(src_p, dst_p, typ_p)
        h = jnp.transpose(x.reshape(n, 2, dh), (1, 0, 2))
        for l in range(num_l):
            wall = jnp.concatenate([w_rel[l], w_root[l][None]], axis=0)
            yrel, root = trans(h, wall, bias[l].reshape(1, d))
            ytab = yrel.reshape(2 * r * n, dh)
            h = agg(ytab, root, gboth, dst_p, w)
        return h

    h1 = branch(x1, edge_index1, edge_type1, W_rel1, W_root1, b1)
    h2 = branch(x2, edge_index2, edge_type2, W_rel2, W_root2, b2)

    nb = -((-n) // bn)
    bpad = nb * bn - n
    b1_3d = jnp.pad(batch1, (0, bpad), constant_values=g).reshape(nb, 1, bn)
    b2_3d = jnp.pad(batch2, (0, bpad), constant_values=g).reshape(nb, 1, bn)
    wm3p = jnp.pad(Wm3, ((0, 0), (0, 128 - out_dim)))
    bm3p = jnp.pad(bm3, (0, 128 - out_dim)).reshape(1, 128)

    pool = _pool_mlp(n, d, g, h_dim, bn)
    out = pool(h1, h2, b1_3d, b2_3d, Wm1, bm1.reshape(1, h_dim),
               Wm2, bm2.reshape(1, h_dim), wm3p, bm3p)
    return out[:, :out_dim]
